# baseline (speedup 1.0000x reference)
import jax
import jax.numpy as jnp
from jax.experimental import pallas as pl
from jax.experimental.pallas import tpu as pltpu

M = 16384
K = 4096
N = 4096

BM = 256
WCHUNK = 256


def _mm_kernel(x_ref, w_hbm_ref, o_ref, w16_ref, bounce_ref, sems_ref):
    nc = N // WCHUNK
    is_first = pl.program_id(0) == 0
    xb = x_ref[...].astype(jnp.bfloat16)

    def _dot(w16):
        return jax.lax.dot_general(
            xb,
            w16,
            dimension_numbers=(((1,), (1,)), ((), ())),
            preferred_element_type=jnp.float32,
        )

    @pl.when(is_first)
    def _load_w_and_compute():
        def _copy(c, slot):
            return pltpu.make_async_copy(
                w_hbm_ref.at[pl.ds(c * WCHUNK, WCHUNK), :],
                bounce_ref.at[slot],
                sems_ref.at[slot],
            )

        _copy(0, 0).start()
        _copy(1, 1).start()

        def _step(c, slot):
            _copy(c, slot).wait()

            @pl.when(c + 2 < nc)
            def _():
                _copy(c + 2, slot).start()

            w16_ref[pl.ds(c * WCHUNK, WCHUNK), :] = bounce_ref[
                slot
            ].astype(jnp.bfloat16)

        def _body(i, carry):
            _step(2 * i, 0)
            _step(2 * i + 1, 1)
            return carry

        jax.lax.fori_loop(0, nc // 4, _body, 0)
        o_ref[:, : N // 2] = _dot(w16_ref[: N // 2, :])
        jax.lax.fori_loop(nc // 4, nc // 2, _body, 0)
        o_ref[:, N // 2 :] = _dot(w16_ref[N // 2 :, :])

    @pl.when(jnp.logical_not(is_first))
    def _compute():
        o_ref[...] = _dot(w16_ref[...])


def kernel(x, W):
    return pl.pallas_call(
        _mm_kernel,
        grid=(M // BM,),
        in_specs=[
            pl.BlockSpec((BM, K), lambda i: (i, 0)),
            pl.BlockSpec(memory_space=pl.ANY),
        ],
        out_specs=pl.BlockSpec((BM, N), lambda i: (i, 0)),
        out_shape=jax.ShapeDtypeStruct((M, N), jnp.float32),
        scratch_shapes=[
            pltpu.VMEM((N, K), jnp.bfloat16),
            pltpu.VMEM((2, WCHUNK, K), jnp.float32),
            pltpu.SemaphoreType.DMA((2,)),
        ],
        compiler_params=pltpu.CompilerParams(
            vmem_limit_bytes=128 * 1024 * 1024,
        ),
    )(x, W)


# speedup vs baseline: 1.0158x; 1.0158x over previous
import jax
import jax.numpy as jnp
from jax.experimental import pallas as pl
from jax.experimental.pallas import tpu as pltpu

M = 16384
K = 4096
N = 4096

BM = 256
WCHUNK = 256


def _mm_kernel(x_ref, w_hbm_ref, o_ref, w16_ref, bounce_ref, sems_ref):
    @pl.when(pl.program_id(0) == 0)
    def _load_w():
        nc = N // WCHUNK

        def _copy(c, slot):
            return pltpu.make_async_copy(
                w_hbm_ref.at[pl.ds(c * WCHUNK, WCHUNK), :],
                bounce_ref.at[slot],
                sems_ref.at[slot],
            )

        _copy(0, 0).start()
        _copy(1, 1).start()

        def _step(c, slot):
            _copy(c, slot).wait()

            @pl.when(c + 2 < nc)
            def _():
                _copy(c + 2, slot).start()

            w16_ref[pl.ds(c * WCHUNK, WCHUNK), :] = bounce_ref[
                slot
            ].astype(jnp.bfloat16)

        def _body(i, carry):
            _step(2 * i, 0)
            _step(2 * i + 1, 1)
            return carry

        jax.lax.fori_loop(0, nc // 2, _body, 0)

    o_ref[...] = jax.lax.dot_general(
        x_ref[...].astype(jnp.bfloat16),
        w16_ref[...],
        dimension_numbers=(((1,), (1,)), ((), ())),
        preferred_element_type=jnp.float32,
    )


def kernel(x, W):
    return pl.pallas_call(
        _mm_kernel,
        grid=(M // BM,),
        in_specs=[
            pl.BlockSpec((BM, K), lambda i: (i, 0)),
            pl.BlockSpec(memory_space=pl.ANY),
        ],
        out_specs=pl.BlockSpec((BM, N), lambda i: (i, 0)),
        out_shape=jax.ShapeDtypeStruct((M, N), jnp.float32),
        scratch_shapes=[
            pltpu.VMEM((N, K), jnp.bfloat16),
            pltpu.VMEM((2, WCHUNK, K), jnp.float32),
            pltpu.SemaphoreType.DMA((2,)),
        ],
        compiler_params=pltpu.CompilerParams(
            vmem_limit_bytes=128 * 1024 * 1024,
        ),
    )(x, W)
